# Initial kernel scaffold; baseline (speedup 1.0000x reference)
import jax
import jax.numpy as jnp
from jax.experimental import pallas as pl


def kernel(x, edge_index, batch, emb, W1, b1, W2, b2, eps, Wc1, bc1, Wc2, bc2):
    raise NotImplementedError("write your pallas kernel here")


# speedup vs baseline: 4.0036x; 4.0036x over previous
import functools

import jax
import jax.numpy as jnp
from jax import lax
from jax.experimental import pallas as pl
from jax.experimental.pallas import tpu as pltpu
from jax.experimental.pallas import tpu_sc as plsc

NC = 2
NS = 16
NW = NC * NS
G = 64
ECH = 128


def _round_up(a, m):
    return (a + m - 1) // m * m


def _emb_gather(emb, idx, n_pad, d):
    rpw = n_pad // NW
    ch = 80 if rpw % 80 == 0 else 64
    nch = rpw // ch
    mesh = plsc.VectorSubcoreMesh(core_axis_name="c", subcore_axis_name="s", num_cores=NC, num_subcores=NS)

    @functools.partial(
        pl.kernel,
        out_type=jax.ShapeDtypeStruct((n_pad, d), jnp.float32),
        mesh=mesh,
        scratch_types=[
            pltpu.VMEM((ch,), jnp.int32),
            pltpu.VMEM((ch, d), jnp.float32),
        ],
    )
    def k(emb_hbm, idx_hbm, h_hbm, idxv, rows):
        wid = lax.axis_index("c") * NS + lax.axis_index("s")
        base = wid * rpw
        for c in range(nch):
            off = base + c * ch
            pltpu.sync_copy(idx_hbm.at[pl.ds(off, ch)], idxv)
            pltpu.sync_copy(emb_hbm.at[idxv], rows)
            pltpu.sync_copy(rows, h_hbm.at[pl.ds(off, ch)])

    return k(emb, idx)


def _edge_agg(h_pad, src_pad, dst_pad, n_pad, d):
    e_pad = src_pad.shape[0]
    epw = e_pad // NW
    nch = epw // ECH
    rps = n_pad // NS
    mesh = plsc.VectorSubcoreMesh(core_axis_name="c", subcore_axis_name="s", num_cores=NC, num_subcores=NS)

    @functools.partial(
        pl.kernel,
        out_type=jax.ShapeDtypeStruct((NC, n_pad, d), jnp.float32),
        mesh=mesh,
        scratch_types=[
            pltpu.VMEM((ECH,), jnp.int32),
            pltpu.VMEM((ECH,), jnp.int32),
            pltpu.VMEM((ECH, d), jnp.float32),
            pltpu.VMEM((ECH, d), jnp.float32),
            pltpu.VMEM_SHARED((n_pad, d), jnp.float32),
        ],
    )
    def k(h_hbm, src_hbm, dst_hbm, out_hbm, srcv, dstv, rows, zbuf, aggsh):
        cid = lax.axis_index("c")
        sid = lax.axis_index("s")
        zv = jnp.zeros((16,), jnp.float32)

        @pl.loop(0, ECH)
        def _(i):
            for j in range(d // 16):
                zbuf[i, pl.ds(j * 16, 16)] = zv

        for r in range(rps // ECH):
            pltpu.sync_copy(zbuf, aggsh.at[pl.ds(sid * rps + r * ECH, ECH)])
        plsc.subcore_barrier()

        base = (cid * NS + sid) * epw

        @pl.loop(0, nch)
        def _(c):
            off = base + c * ECH
            pltpu.sync_copy(src_hbm.at[pl.ds(off, ECH)], srcv)
            pltpu.sync_copy(dst_hbm.at[pl.ds(off, ECH)], dstv)
            pltpu.sync_copy(h_hbm.at[srcv], rows)
            pltpu.sync_copy(rows, aggsh.at[dstv], add=True)

        plsc.subcore_barrier()
        pltpu.sync_copy(aggsh.at[pl.ds(sid * rps, rps)],
                        out_hbm.at[cid, pl.ds(sid * rps, rps)])

    return k(h_pad, src_pad, dst_pad)


def _mlp_body(h_ref, a_ref, w1_ref, b1_ref, w2_ref, b2_ref, sc_ref, o_ref):
    s = sc_ref[0, 0]
    z = h_ref[...] * s + a_ref[0] + a_ref[1]
    z = jnp.maximum(
        jnp.dot(z, w1_ref[...], preferred_element_type=jnp.float32)
        + b1_ref[...], 0.0)
    z = jnp.maximum(
        jnp.dot(z, w2_ref[...], preferred_element_type=jnp.float32)
        + b2_ref[...], 0.0)
    o_ref[...] = z


def _mlp(h_pad, agg2, w1, b1, w2, b2, scale, n_pad, d, blk):
    nb = n_pad // blk
    return pl.pallas_call(
        _mlp_body,
        grid=(nb,),
        in_specs=[
            pl.BlockSpec((blk, d), lambda i: (i, 0)),
            pl.BlockSpec((NC, blk, d), lambda i: (0, i, 0)),
            pl.BlockSpec((d, d), lambda i: (0, 0)),
            pl.BlockSpec((1, d), lambda i: (0, 0)),
            pl.BlockSpec((d, d), lambda i: (0, 0)),
            pl.BlockSpec((1, d), lambda i: (0, 0)),
            pl.BlockSpec(memory_space=pltpu.SMEM),
        ],
        out_specs=pl.BlockSpec((blk, d), lambda i: (i, 0)),
        out_shape=jax.ShapeDtypeStruct((n_pad, d), jnp.float32),
    )(h_pad, agg2, w1, b1, w2, b2, scale)


def _mlp_pool_cls(h_pad, agg2, w1, b1, w2, b2, scale, batch3, wc1, bc1,
                  wc2, bc2, n_pad, d, blk):
    nb = n_pad // blk

    def body(h_ref, a_ref, w1_ref, b1_ref, w2_ref, b2_ref, sc_ref, bt_ref,
             wc1_ref, bc1_ref, wc2_ref, bc2_ref, s_ref, hg_ref):
        i = pl.program_id(0)
        s = sc_ref[0, 0]
        z = h_ref[...] * s + a_ref[0] + a_ref[1]
        z = jnp.maximum(
            jnp.dot(z, w1_ref[...], preferred_element_type=jnp.float32)
            + b1_ref[...], 0.0)
        z = jnp.maximum(
            jnp.dot(z, w2_ref[...], preferred_element_type=jnp.float32)
            + b2_ref[...], 0.0)
        bt = bt_ref[0, 0, :]
        oh = (bt[:, None] == lax.broadcasted_iota(jnp.int32, (blk, G), 1))
        oh = oh.astype(jnp.float32)
        contrib = lax.dot_general(
            oh, z, (((0,), (0,)), ((), ())),
            preferred_element_type=jnp.float32)

        @pl.when(i == 0)
        def _():
            hg_ref[...] = jnp.zeros_like(hg_ref)

        hg_ref[...] += contrib

        @pl.when(i == nb - 1)
        def _():
            hg = hg_ref[...]
            hid = jnp.maximum(
                jnp.dot(hg, wc1_ref[...], preferred_element_type=jnp.float32)
                + bc1_ref[...], 0.0)
            logit = jnp.dot(hid, wc2_ref[...],
                            preferred_element_type=jnp.float32) + bc2_ref[0, 0]
            s_ref[...] = jax.nn.sigmoid(logit)

    return pl.pallas_call(
        body,
        grid=(nb,),
        in_specs=[
            pl.BlockSpec((blk, d), lambda i: (i, 0)),
            pl.BlockSpec((NC, blk, d), lambda i: (0, i, 0)),
            pl.BlockSpec((d, d), lambda i: (0, 0)),
            pl.BlockSpec((1, d), lambda i: (0, 0)),
            pl.BlockSpec((d, d), lambda i: (0, 0)),
            pl.BlockSpec((1, d), lambda i: (0, 0)),
            pl.BlockSpec(memory_space=pltpu.SMEM),
            pl.BlockSpec((1, 1, blk), lambda i: (i, 0, 0)),
            pl.BlockSpec((d, d), lambda i: (0, 0)),
            pl.BlockSpec((1, d), lambda i: (0, 0)),
            pl.BlockSpec((d, 1), lambda i: (0, 0)),
            pl.BlockSpec(memory_space=pltpu.SMEM),
        ],
        out_specs=pl.BlockSpec((G, 1), lambda i: (0, 0)),
        out_shape=jax.ShapeDtypeStruct((G, 1), jnp.float32),
        scratch_shapes=[pltpu.VMEM((G, d), jnp.float32)],
    )(h_pad, agg2, w1, b1, w2, b2, scale, batch3, wc1, bc1, wc2, bc2)


def kernel(x, edge_index, batch, emb, W1, b1, W2, b2, eps, Wc1, bc1, Wc2, bc2):
    n = x.shape[0]
    d = emb.shape[1]
    e = edge_index.shape[1]
    n_layers = W1.shape[0]
    blk = 512

    n_pad = _round_up(n + 1, NW * 80)
    e_pad = _round_up(e, NW * ECH)

    idx = jnp.concatenate(
        [x[:, 0], jnp.zeros((n_pad - n,), jnp.int32)])
    src_pad = jnp.concatenate(
        [edge_index[0], jnp.full((e_pad - e,), n, jnp.int32)])
    dst_pad = jnp.concatenate(
        [edge_index[1], jnp.full((e_pad - e,), n, jnp.int32)])
    batch3 = jnp.concatenate(
        [batch, jnp.full((n_pad - n,), G, jnp.int32)]).reshape(
            n_pad // blk, 1, blk)

    h = _emb_gather(emb, idx, n_pad, d)
    for l in range(n_layers):
        agg2 = _edge_agg(h, src_pad, dst_pad, n_pad, d)
        scale = (1.0 + eps[l]).reshape(1, 1)
        b1l = b1[l].reshape(1, d)
        b2l = b2[l].reshape(1, d)
        if l < n_layers - 1:
            h = _mlp(h, agg2, W1[l], b1l, W2[l], b2l, scale, n_pad, d, blk)
        else:
            score = _mlp_pool_cls(
                h, agg2, W1[l], b1l, W2[l], b2l, scale, batch3,
                Wc1, bc1.reshape(1, d), Wc2, bc2.reshape(1, 1),
                n_pad, d, blk)
    return score.reshape(-1)
